# baseline (speedup 1.0000x reference)
import functools

import jax
import jax.numpy as jnp
import numpy as np
from jax import lax
from jax.experimental import pallas as pl
from jax.experimental.pallas import tpu as pltpu
from jax.experimental.pallas import tpu_sc as plsc

N = 10000
NP = 10240
F = 128
F2 = F // 2
E = 320000
ETOT = E + N
NC, NS = 2, 16
NW = NC * NS
CHB = 128
CHD = 81
CH = 162
EP = NS * CH * CHB
PAD_E = EP - ETOT
ZONE = NP // NS
NBUF = 9
ROWBLK = 1000
GRID = N // ROWBLK

_PAD = np.arange(PAD_E)
_TAIL_SRC = np.concatenate([np.arange(N), _PAD % N]).astype(np.int32)
_TAIL_DST = np.concatenate([np.arange(N), N + _PAD % (NP - N)]).astype(np.int32)


def _deg_body(dst_hbm, deg_hbm, dst_v, ones_v, zb_v, acc):
    c = lax.axis_index("c")
    s = lax.axis_index("s")
    wid = c * NS + s
    pltpu.sync_copy(dst_hbm.at[wid], dst_v)
    one16 = jnp.ones((16,), jnp.float32)
    zero16 = jnp.zeros((16,), jnp.float32)
    for j in range(CHB // 16):
        ones_v[pl.ds(j * 16, 16)] = one16
    for j in range(ZONE // 16):
        zb_v[pl.ds(j * 16, 16)] = zero16
    pltpu.sync_copy(zb_v, acc.at[pl.ds(s * ZONE, ZONE)])
    plsc.subcore_barrier()

    def chunk(g, carry):
        pltpu.sync_copy(ones_v, acc.at[dst_v.at[g]], add=True)
        return carry

    lax.fori_loop(0, CHD, chunk, 0)
    plsc.subcore_barrier()
    pltpu.sync_copy(acc.at[pl.ds(s * ZONE, ZONE)], deg_hbm.at[c, pl.ds(s * ZONE, ZONE)])


def _agg_body(h_hbm, src_hbm, dst_hbm, zeros_hbm, out_hbm, src_v, dst_v,
              *scr):
    c = lax.axis_index("c")
    s = lax.axis_index("s")
    bufs = scr[:NBUF]
    acc = scr[NBUF]
    sems = scr[NBUF + 1:]
    hc = h_hbm.at[c]
    pltpu.sync_copy(src_hbm.at[s], src_v)
    pltpu.sync_copy(dst_hbm.at[s], dst_v)
    pltpu.sync_copy(zeros_hbm, acc.at[pl.ds(s * ZONE, ZONE)])
    plsc.subcore_barrier()

    for b in range(NBUF):
        pltpu.async_copy(hc.at[src_v.at[b]], bufs[b], sems[b])

    def outer(o, carry):
        for b in range(NBUF):
            g = o * NBUF + b
            pltpu.make_async_copy(hc.at[src_v.at[g]], bufs[b], sems[b]).wait()
            pltpu.sync_copy(bufs[b], acc.at[dst_v.at[g]], add=True)

            @pl.when(g + NBUF < CH)
            def _issue():
                pltpu.async_copy(hc.at[src_v.at[g + NBUF]], bufs[b], sems[b])
        return carry

    lax.fori_loop(0, CH // NBUF, outer, 0)
    plsc.subcore_barrier()
    pltpu.sync_copy(acc.at[pl.ds(s * ZONE, ZONE)],
                    out_hbm.at[c, pl.ds(s * ZONE, ZONE)])


@functools.lru_cache(maxsize=None)
def _sc_kernels():
    mesh = plsc.VectorSubcoreMesh(
        core_axis_name="c", subcore_axis_name="s", num_cores=NC, num_subcores=NS
    )
    deg_kernel = pl.kernel(
        _deg_body,
        out_type=jax.ShapeDtypeStruct((NC, NP), jnp.float32),
        mesh=mesh,
        scratch_types=[
            pltpu.VMEM((CHD, CHB), jnp.int32),
            pltpu.VMEM((CHB,), jnp.float32),
            pltpu.VMEM((ZONE,), jnp.float32),
            pltpu.VMEM_SHARED((NP,), jnp.float32),
        ],
    )
    agg_kernel = pl.kernel(
        _agg_body,
        out_type=jax.ShapeDtypeStruct((NC, NP, F2), jnp.bfloat16),
        mesh=mesh,
        scratch_types=(
            [pltpu.VMEM((CH, CHB), jnp.int32),
             pltpu.VMEM((CH, CHB), jnp.int32)]
            + [pltpu.VMEM((CHB, F2), jnp.bfloat16)] * NBUF
            + [pltpu.VMEM_SHARED((NP, F2), jnp.bfloat16)]
            + [pltpu.SemaphoreType.DMA] * NBUF
        ),
        compiler_params=pltpu.CompilerParams(use_tc_tiling_on_sc=False),
    )
    return deg_kernel, agg_kernel


def _dinv_of(deg_ref):
    deg = deg_ref[0] + deg_ref[1]
    return lax.rsqrt(jnp.maximum(deg, 1.0))


def _mm1_body(x_ref, w_ref, deg_ref, o_ref):
    dinv = _dinv_of(deg_ref)
    h = jnp.dot(x_ref[...], w_ref[...], preferred_element_type=jnp.float32) * dinv
    hb = h.astype(jnp.bfloat16)
    o_ref[0] = hb[:, :F2]
    o_ref[1] = hb[:, F2:]


def _mm2_body(a_ref, deg_ref, b_ref, w_ref, o_ref):
    dinv = _dinv_of(deg_ref)
    a = jnp.concatenate([a_ref[0], a_ref[1]], axis=1).astype(jnp.float32)
    z = jnp.maximum(a * dinv + b_ref[...], 0.0)
    h = jnp.dot(z, w_ref[...], preferred_element_type=jnp.float32) * dinv
    hb = h.astype(jnp.bfloat16)
    o_ref[0] = hb[:, :F2]
    o_ref[1] = hb[:, F2:]


def _fin_body(a_ref, deg_ref, b_ref, o_ref):
    dinv = _dinv_of(deg_ref)
    a = jnp.concatenate([a_ref[0], a_ref[1]], axis=1).astype(jnp.float32)
    o_ref[...] = jnp.maximum(a * dinv + b_ref[...], 0.0)


_rows_spec = pl.BlockSpec((ROWBLK, F), lambda i: (i, 0))
_half_spec = pl.BlockSpec((NC, ROWBLK, F2), lambda i: (0, i, 0))
_deg_spec = pl.BlockSpec((NC, ROWBLK, 1), lambda i: (0, i, 0))
_w_spec = pl.BlockSpec((F, F), lambda i: (0, 0))
_b_spec = pl.BlockSpec((1, F), lambda i: (0, 0))
_out_rows = jax.ShapeDtypeStruct((N, F), jnp.float32)
_out_half = jax.ShapeDtypeStruct((NC, N, F2), jnp.bfloat16)

_mm1 = pl.pallas_call(
    _mm1_body, grid=(GRID,),
    in_specs=[_rows_spec, _w_spec, _deg_spec],
    out_specs=_half_spec, out_shape=_out_half)

_mm2 = pl.pallas_call(
    _mm2_body, grid=(GRID,),
    in_specs=[_half_spec, _deg_spec, _b_spec, _w_spec],
    out_specs=_half_spec, out_shape=_out_half)

_fin = pl.pallas_call(
    _fin_body, grid=(GRID,),
    in_specs=[_half_spec, _deg_spec, _b_spec],
    out_specs=_rows_spec, out_shape=_out_rows)


def kernel(x, edge_index, W1, b1, W2, b2):
    srcf = jnp.concatenate([edge_index[0], _TAIL_SRC])
    dstf = jnp.concatenate([edge_index[1], _TAIL_DST])
    src_a = srcf.reshape(NS, CH, CHB)
    dst_a = dstf.reshape(NS, CH, CHB)
    dst_d = dstf.reshape(NW, CHD, CHB)
    zeros = jnp.zeros((ZONE, F2), jnp.bfloat16)

    deg_kernel, agg_kernel = _sc_kernels()
    deg = deg_kernel(dst_d).reshape(NC, NP, 1)[:, :N]
    h1 = _mm1(x, W1, deg)
    agg1 = agg_kernel(h1, src_a, dst_a, zeros)
    h2 = _mm2(agg1, deg, b1.reshape(1, F), W2)
    agg2 = agg_kernel(h2, src_a, dst_a, zeros)
    return _fin(agg2, deg, b2.reshape(1, F))


# speedup vs baseline: 1.0284x; 1.0284x over previous
import functools

import jax
import jax.numpy as jnp
import numpy as np
from jax import lax
from jax.experimental import pallas as pl
from jax.experimental.pallas import tpu as pltpu
from jax.experimental.pallas import tpu_sc as plsc

N = 10000
NP = 10240
F = 128
F2 = F // 2
E = 320000
ETOT = E + N
NC, NS = 2, 16
NW = NC * NS
CHB = 128
CHD = 81
CH = 162
EP = NS * CH * CHB
PAD_E = EP - ETOT
ZONE = NP // NS
NBUF = 6
ROWBLK = 2000
GRID = N // ROWBLK

_PAD = np.arange(PAD_E)
_TAIL_SRC = np.concatenate([np.arange(N), _PAD % N]).astype(np.int32)
_TAIL_DST = np.concatenate([np.arange(N), N + _PAD % (NP - N)]).astype(np.int32)


def _deg_body(dst_hbm, deg_hbm, dst_v, ones_v, zb_v, acc):
    c = lax.axis_index("c")
    s = lax.axis_index("s")
    wid = c * NS + s
    pltpu.sync_copy(dst_hbm.at[wid], dst_v)
    one16 = jnp.ones((16,), jnp.float32)
    zero16 = jnp.zeros((16,), jnp.float32)
    for j in range(CHB // 16):
        ones_v[pl.ds(j * 16, 16)] = one16
    for j in range(ZONE // 16):
        zb_v[pl.ds(j * 16, 16)] = zero16
    pltpu.sync_copy(zb_v, acc.at[pl.ds(s * ZONE, ZONE)])
    plsc.subcore_barrier()

    def chunk(g, carry):
        pltpu.sync_copy(ones_v, acc.at[dst_v.at[g]], add=True)
        return carry

    lax.fori_loop(0, CHD, chunk, 0)
    plsc.subcore_barrier()
    pltpu.sync_copy(acc.at[pl.ds(s * ZONE, ZONE)], deg_hbm.at[c, pl.ds(s * ZONE, ZONE)])


def _agg_body(h_hbm, src_hbm, dst_hbm, zeros_hbm, out_hbm, src_v, dst_v,
              *scr):
    c = lax.axis_index("c")
    s = lax.axis_index("s")
    bufs = scr[:NBUF]
    acc = scr[NBUF]
    sems = scr[NBUF + 1:]
    hc = h_hbm.at[c]
    pltpu.sync_copy(src_hbm.at[s], src_v)
    pltpu.sync_copy(dst_hbm.at[s], dst_v)
    pltpu.sync_copy(zeros_hbm, acc.at[pl.ds(s * ZONE, ZONE)])
    plsc.subcore_barrier()

    for b in range(NBUF):
        pltpu.async_copy(hc.at[src_v.at[b]], bufs[b], sems[b])

    def outer(o, carry):
        for b in range(NBUF):
            g = o * NBUF + b
            pltpu.make_async_copy(hc.at[src_v.at[g]], bufs[b], sems[b]).wait()
            pltpu.sync_copy(bufs[b], acc.at[dst_v.at[g]], add=True)

            @pl.when(g + NBUF < CH)
            def _issue():
                pltpu.async_copy(hc.at[src_v.at[g + NBUF]], bufs[b], sems[b])
        return carry

    lax.fori_loop(0, CH // NBUF, outer, 0)
    plsc.subcore_barrier()
    pltpu.sync_copy(acc.at[pl.ds(s * ZONE, ZONE)],
                    out_hbm.at[c, pl.ds(s * ZONE, ZONE)])


@functools.lru_cache(maxsize=None)
def _sc_kernels():
    mesh = plsc.VectorSubcoreMesh(
        core_axis_name="c", subcore_axis_name="s", num_cores=NC, num_subcores=NS
    )
    deg_kernel = pl.kernel(
        _deg_body,
        out_type=jax.ShapeDtypeStruct((NC, NP), jnp.float32),
        mesh=mesh,
        scratch_types=[
            pltpu.VMEM((CHD, CHB), jnp.int32),
            pltpu.VMEM((CHB,), jnp.float32),
            pltpu.VMEM((ZONE,), jnp.float32),
            pltpu.VMEM_SHARED((NP,), jnp.float32),
        ],
    )
    agg_kernel = pl.kernel(
        _agg_body,
        out_type=jax.ShapeDtypeStruct((NC, NP, F2), jnp.bfloat16),
        mesh=mesh,
        scratch_types=(
            [pltpu.VMEM((CH, CHB), jnp.int32),
             pltpu.VMEM((CH, CHB), jnp.int32)]
            + [pltpu.VMEM((CHB, F2), jnp.bfloat16)] * NBUF
            + [pltpu.VMEM_SHARED((NP, F2), jnp.bfloat16)]
            + [pltpu.SemaphoreType.DMA] * NBUF
        ),
        compiler_params=pltpu.CompilerParams(use_tc_tiling_on_sc=False),
    )
    return deg_kernel, agg_kernel


def _dinv_of(deg_ref):
    deg = deg_ref[0] + deg_ref[1]
    return lax.rsqrt(jnp.maximum(deg, 1.0))


def _mm1_body(x_ref, w_ref, deg_ref, o_ref):
    dinv = _dinv_of(deg_ref)
    h = jnp.dot(x_ref[...], w_ref[...], preferred_element_type=jnp.float32) * dinv
    hb = h.astype(jnp.bfloat16)
    o_ref[0] = hb[:, :F2]
    o_ref[1] = hb[:, F2:]


def _mm2_body(a_ref, deg_ref, b_ref, w_ref, o_ref):
    dinv = _dinv_of(deg_ref)
    a = jnp.concatenate([a_ref[0], a_ref[1]], axis=1).astype(jnp.float32)
    z = jnp.maximum(a * dinv + b_ref[...], 0.0)
    h = jnp.dot(z, w_ref[...], preferred_element_type=jnp.float32) * dinv
    hb = h.astype(jnp.bfloat16)
    o_ref[0] = hb[:, :F2]
    o_ref[1] = hb[:, F2:]


def _fin_body(a_ref, deg_ref, b_ref, o_ref):
    dinv = _dinv_of(deg_ref)
    a = jnp.concatenate([a_ref[0], a_ref[1]], axis=1).astype(jnp.float32)
    o_ref[...] = jnp.maximum(a * dinv + b_ref[...], 0.0)


_rows_spec = pl.BlockSpec((ROWBLK, F), lambda i: (i, 0))
_half_spec = pl.BlockSpec((NC, ROWBLK, F2), lambda i: (0, i, 0))
_deg_spec = pl.BlockSpec((NC, ROWBLK, 1), lambda i: (0, i, 0))
_w_spec = pl.BlockSpec((F, F), lambda i: (0, 0))
_b_spec = pl.BlockSpec((1, F), lambda i: (0, 0))
_out_rows = jax.ShapeDtypeStruct((N, F), jnp.float32)
_out_half = jax.ShapeDtypeStruct((NC, N, F2), jnp.bfloat16)

_mm1 = pl.pallas_call(
    _mm1_body, grid=(GRID,),
    in_specs=[_rows_spec, _w_spec, _deg_spec],
    out_specs=_half_spec, out_shape=_out_half)

_mm2 = pl.pallas_call(
    _mm2_body, grid=(GRID,),
    in_specs=[_half_spec, _deg_spec, _b_spec, _w_spec],
    out_specs=_half_spec, out_shape=_out_half)

_fin = pl.pallas_call(
    _fin_body, grid=(GRID,),
    in_specs=[_half_spec, _deg_spec, _b_spec],
    out_specs=_rows_spec, out_shape=_out_rows)


def kernel(x, edge_index, W1, b1, W2, b2):
    srcf = jnp.concatenate([edge_index[0], _TAIL_SRC])
    dstf = jnp.concatenate([edge_index[1], _TAIL_DST])
    src_a = srcf.reshape(NS, CH, CHB)
    dst_a = dstf.reshape(NS, CH, CHB)
    dst_d = dstf.reshape(NW, CHD, CHB)
    zeros = jnp.zeros((ZONE, F2), jnp.bfloat16)

    deg_kernel, agg_kernel = _sc_kernels()
    deg = deg_kernel(dst_d).reshape(NC, NP, 1)[:, :N]
    h1 = _mm1(x, W1, deg)
    agg1 = agg_kernel(h1, src_a, dst_a, zeros)
    h2 = _mm2(agg1, deg, b1.reshape(1, F), W2)
    agg2 = agg_kernel(h2, src_a, dst_a, zeros)
    return _fin(agg2, deg, b2.reshape(1, F))


# speedup vs baseline: 1.0301x; 1.0017x over previous
import functools

import jax
import jax.numpy as jnp
import numpy as np
from jax import lax
from jax.experimental import pallas as pl
from jax.experimental.pallas import tpu as pltpu
from jax.experimental.pallas import tpu_sc as plsc

N = 10000
NP = 10240
F = 128
F2 = F // 2
E = 320000
ETOT = E + N
NC, NS = 2, 16
NW = NC * NS
CHB = 128
CHD = 81
CH = 162
EP = NS * CH * CHB
PAD_E = EP - ETOT
ZONE = NP // NS
NBUF = 6
ROWBLK = 2000
GRID = N // ROWBLK

_PAD = np.arange(PAD_E)
_TAIL_SRC = np.concatenate([np.arange(N), _PAD % N]).astype(np.int32)
_TAIL_DST = np.concatenate([np.arange(N), N + _PAD % (NP - N)]).astype(np.int32)


def _deg_body(dst_hbm, deg_hbm, dst_v, ones_v, zb_v, acc):
    c = lax.axis_index("c")
    s = lax.axis_index("s")
    wid = c * NS + s
    pltpu.sync_copy(dst_hbm.at[wid], dst_v)
    one16 = jnp.ones((16,), jnp.float32)
    zero16 = jnp.zeros((16,), jnp.float32)
    for j in range(CHB // 16):
        ones_v[pl.ds(j * 16, 16)] = one16
    for j in range(ZONE // 16):
        zb_v[pl.ds(j * 16, 16)] = zero16
    pltpu.sync_copy(zb_v, acc.at[pl.ds(s * ZONE, ZONE)])
    plsc.subcore_barrier()

    def chunk(g, carry):
        pltpu.sync_copy(ones_v, acc.at[dst_v.at[g]], add=True)
        return carry

    lax.fori_loop(0, CHD, chunk, 0)
    plsc.subcore_barrier()
    pltpu.sync_copy(acc.at[pl.ds(s * ZONE, ZONE)], deg_hbm.at[c, pl.ds(s * ZONE, ZONE)])


def _agg_body(h_hbm, src_hbm, dst_hbm, zeros_hbm, out_hbm, src_v, dst_v,
              *scr):
    c = lax.axis_index("c")
    s = lax.axis_index("s")
    bufs = scr[:NBUF]
    acc = scr[NBUF]
    sems = scr[NBUF + 1:]
    hc = h_hbm.at[c]
    pltpu.sync_copy(src_hbm.at[s], src_v)
    pltpu.sync_copy(dst_hbm.at[s], dst_v)
    pltpu.sync_copy(zeros_hbm, acc.at[pl.ds(s * ZONE, ZONE)])
    plsc.subcore_barrier()

    for b in range(NBUF):
        pltpu.async_copy(hc.at[src_v.at[b]], bufs[b], sems[b])

    def outer(o, carry):
        for b in range(NBUF):
            g = o * NBUF + b
            pltpu.make_async_copy(hc.at[src_v.at[g]], bufs[b], sems[b]).wait()
            pltpu.sync_copy(bufs[b], acc.at[dst_v.at[g]], add=True)

            @pl.when(g + NBUF < CH)
            def _issue():
                pltpu.async_copy(hc.at[src_v.at[g + NBUF]], bufs[b], sems[b])
        return carry

    lax.fori_loop(0, CH // NBUF, outer, 0)
    plsc.subcore_barrier()
    pltpu.sync_copy(acc.at[pl.ds(s * ZONE, ZONE)],
                    out_hbm.at[c, pl.ds(s * ZONE, ZONE)])


@functools.lru_cache(maxsize=None)
def _sc_kernels():
    mesh = plsc.VectorSubcoreMesh(
        core_axis_name="c", subcore_axis_name="s", num_cores=NC, num_subcores=NS
    )
    deg_kernel = pl.kernel(
        _deg_body,
        out_type=jax.ShapeDtypeStruct((NC, NP), jnp.float32),
        mesh=mesh,
        scratch_types=[
            pltpu.VMEM((CHD, CHB), jnp.int32),
            pltpu.VMEM((CHB,), jnp.float32),
            pltpu.VMEM((ZONE,), jnp.float32),
            pltpu.VMEM_SHARED((NP,), jnp.float32),
        ],
    )
    agg_kernel = pl.kernel(
        _agg_body,
        out_type=jax.ShapeDtypeStruct((NC, NP, F2), jnp.bfloat16),
        mesh=mesh,
        scratch_types=(
            [pltpu.VMEM((CH, CHB), jnp.int32),
             pltpu.VMEM((CH, CHB), jnp.int32)]
            + [pltpu.VMEM((CHB, F2), jnp.bfloat16)] * NBUF
            + [pltpu.VMEM_SHARED((NP, F2), jnp.bfloat16)]
            + [pltpu.SemaphoreType.DMA] * NBUF
        ),
        compiler_params=pltpu.CompilerParams(use_tc_tiling_on_sc=False),
    )
    return deg_kernel, agg_kernel


def _dinv_of(deg_ref):
    deg = deg_ref[0] + deg_ref[1]
    return lax.rsqrt(jnp.maximum(deg, 1.0))


def _mma_body(x_ref, w_ref, o_ref):
    o_ref[...] = jnp.dot(x_ref[...], w_ref[...],
                         preferred_element_type=jnp.float32)


def _scale_body(h_ref, deg_ref, o_ref):
    dinv = _dinv_of(deg_ref)
    hb = (h_ref[...] * dinv).astype(jnp.bfloat16)
    o_ref[0] = hb[:, :F2]
    o_ref[1] = hb[:, F2:]


def _mm2_body(a_ref, deg_ref, b_ref, w_ref, o_ref):
    dinv = _dinv_of(deg_ref)
    a = jnp.concatenate([a_ref[0], a_ref[1]], axis=1).astype(jnp.float32)
    z = jnp.maximum(a * dinv + b_ref[...], 0.0)
    h = jnp.dot(z, w_ref[...], preferred_element_type=jnp.float32) * dinv
    hb = h.astype(jnp.bfloat16)
    o_ref[0] = hb[:, :F2]
    o_ref[1] = hb[:, F2:]


def _fin_body(a_ref, deg_ref, b_ref, o_ref):
    dinv = _dinv_of(deg_ref)
    a = jnp.concatenate([a_ref[0], a_ref[1]], axis=1).astype(jnp.float32)
    o_ref[...] = jnp.maximum(a * dinv + b_ref[...], 0.0)


_rows_spec = pl.BlockSpec((ROWBLK, F), lambda i: (i, 0))
_half_spec = pl.BlockSpec((NC, ROWBLK, F2), lambda i: (0, i, 0))
_deg_spec = pl.BlockSpec((NC, ROWBLK, 1), lambda i: (0, i, 0))
_w_spec = pl.BlockSpec((F, F), lambda i: (0, 0))
_b_spec = pl.BlockSpec((1, F), lambda i: (0, 0))
_out_rows = jax.ShapeDtypeStruct((N, F), jnp.float32)
_out_half = jax.ShapeDtypeStruct((NC, N, F2), jnp.bfloat16)

_mma = pl.pallas_call(
    _mma_body, grid=(GRID,),
    in_specs=[_rows_spec, _w_spec],
    out_specs=_rows_spec, out_shape=_out_rows)

_scale = pl.pallas_call(
    _scale_body, grid=(GRID,),
    in_specs=[_rows_spec, _deg_spec],
    out_specs=_half_spec, out_shape=_out_half)

_mm2 = pl.pallas_call(
    _mm2_body, grid=(GRID,),
    in_specs=[_half_spec, _deg_spec, _b_spec, _w_spec],
    out_specs=_half_spec, out_shape=_out_half)

_fin = pl.pallas_call(
    _fin_body, grid=(GRID,),
    in_specs=[_half_spec, _deg_spec, _b_spec],
    out_specs=_rows_spec, out_shape=_out_rows)


def kernel(x, edge_index, W1, b1, W2, b2):
    srcf = jnp.concatenate([edge_index[0], _TAIL_SRC])
    dstf = jnp.concatenate([edge_index[1], _TAIL_DST])
    src_a = srcf.reshape(NS, CH, CHB)
    dst_a = dstf.reshape(NS, CH, CHB)
    dst_d = dstf.reshape(NW, CHD, CHB)
    zeros = jnp.zeros((ZONE, F2), jnp.bfloat16)

    deg_kernel, agg_kernel = _sc_kernels()
    xw1 = _mma(x, W1)
    deg = deg_kernel(dst_d).reshape(NC, NP, 1)[:, :N]
    h1 = _scale(xw1, deg)
    agg1 = agg_kernel(h1, src_a, dst_a, zeros)
    h2 = _mm2(agg1, deg, b1.reshape(1, F), W2)
    agg2 = agg_kernel(h2, src_a, dst_a, zeros)
    return _fin(agg2, deg, b2.reshape(1, F))


# speedup vs baseline: 1.0456x; 1.0151x over previous
import functools

import jax
import jax.numpy as jnp
import numpy as np
from jax import lax
from jax.experimental import pallas as pl
from jax.experimental.pallas import tpu as pltpu
from jax.experimental.pallas import tpu_sc as plsc

N = 10000
NP = 10240
F = 128
F2 = F // 2
E = 320000
ETOT = E + N
NC, NS = 2, 16
NW = NC * NS
CHB = 128
CHD = 81
CH = 162
EP = NS * CH * CHB
PAD_E = EP - ETOT
ZONE = NP // NS
NBUF = 6
ROWBLK = 2000
GRID = N // ROWBLK

_PAD = np.arange(PAD_E)
_TAIL_SRC = np.concatenate([np.arange(N), _PAD % N]).astype(np.int32)
_TAIL_DST = np.concatenate([np.arange(N), N + _PAD % (NP - N)]).astype(np.int32)


def _deg_body(dst_hbm, deg_hbm, dst_v, ones_v, zb_v, acc, dsem):
    c = lax.axis_index("c")
    s = lax.axis_index("s")
    wid = c * NS + s
    pltpu.sync_copy(dst_hbm.at[wid], dst_v)
    one16 = jnp.ones((16,), jnp.float32)
    zero16 = jnp.zeros((16,), jnp.float32)
    for j in range(CHB // 16):
        ones_v[pl.ds(j * 16, 16)] = one16
    for j in range(ZONE // 16):
        zb_v[pl.ds(j * 16, 16)] = zero16
    pltpu.sync_copy(zb_v, acc.at[pl.ds(s * ZONE, ZONE)])
    plsc.subcore_barrier()

    def chunk(g, carry):
        pltpu.async_copy(ones_v, acc.at[dst_v.at[g]], dsem, add=True)
        return carry

    lax.fori_loop(0, CHD, chunk, 0)

    def chunk_wait(g, carry):
        pltpu.make_async_copy(ones_v, acc.at[dst_v.at[g]], dsem).wait()
        return carry

    lax.fori_loop(0, CHD, chunk_wait, 0)
    plsc.subcore_barrier()
    pltpu.sync_copy(acc.at[pl.ds(s * ZONE, ZONE)], deg_hbm.at[c, pl.ds(s * ZONE, ZONE)])


def _agg_body(h_hbm, src_hbm, dst_hbm, zeros_hbm, out_hbm, src_v, dst_v,
              *scr):
    c = lax.axis_index("c")
    s = lax.axis_index("s")
    bufs = scr[:NBUF]
    acc = scr[NBUF]
    sems = scr[NBUF + 1:]
    hc = h_hbm.at[c]
    pltpu.sync_copy(src_hbm.at[s], src_v)
    pltpu.sync_copy(dst_hbm.at[s], dst_v)
    pltpu.sync_copy(zeros_hbm, acc.at[pl.ds(s * ZONE, ZONE)])
    plsc.subcore_barrier()

    for b in range(NBUF):
        pltpu.async_copy(hc.at[src_v.at[b]], bufs[b], sems[b])

    def outer(o, carry):
        for b in range(NBUF):
            g = o * NBUF + b
            pltpu.make_async_copy(hc.at[src_v.at[g]], bufs[b], sems[b]).wait()
            pltpu.sync_copy(bufs[b], acc.at[dst_v.at[g]], add=True)

            @pl.when(g + NBUF < CH)
            def _issue():
                pltpu.async_copy(hc.at[src_v.at[g + NBUF]], bufs[b], sems[b])
        return carry

    lax.fori_loop(0, CH // NBUF, outer, 0)
    plsc.subcore_barrier()
    pltpu.sync_copy(acc.at[pl.ds(s * ZONE, ZONE)],
                    out_hbm.at[c, pl.ds(s * ZONE, ZONE)])


@functools.lru_cache(maxsize=None)
def _sc_kernels():
    mesh = plsc.VectorSubcoreMesh(
        core_axis_name="c", subcore_axis_name="s", num_cores=NC, num_subcores=NS
    )
    deg_kernel = pl.kernel(
        _deg_body,
        out_type=jax.ShapeDtypeStruct((NC, NP), jnp.float32),
        mesh=mesh,
        scratch_types=[
            pltpu.VMEM((CHD, CHB), jnp.int32),
            pltpu.VMEM((CHB,), jnp.float32),
            pltpu.VMEM((ZONE,), jnp.float32),
            pltpu.VMEM_SHARED((NP,), jnp.float32),
            pltpu.SemaphoreType.DMA,
        ],
    )
    agg_kernel = pl.kernel(
        _agg_body,
        out_type=jax.ShapeDtypeStruct((NC, NP, F2), jnp.bfloat16),
        mesh=mesh,
        scratch_types=(
            [pltpu.VMEM((CH, CHB), jnp.int32),
             pltpu.VMEM((CH, CHB), jnp.int32)]
            + [pltpu.VMEM((CHB, F2), jnp.bfloat16)] * NBUF
            + [pltpu.VMEM_SHARED((NP, F2), jnp.bfloat16)]
            + [pltpu.SemaphoreType.DMA] * NBUF
        ),
        compiler_params=pltpu.CompilerParams(use_tc_tiling_on_sc=False),
    )
    return deg_kernel, agg_kernel


def _dinv_of(deg_ref):
    deg = deg_ref[0] + deg_ref[1]
    return lax.rsqrt(jnp.maximum(deg, 1.0))


def _mma_body(x_ref, w_ref, o_ref):
    o_ref[...] = jnp.dot(x_ref[...], w_ref[...],
                         preferred_element_type=jnp.float32)


def _scale_body(h_ref, deg_ref, o_ref):
    dinv = _dinv_of(deg_ref)
    hb = (h_ref[...] * dinv).astype(jnp.bfloat16)
    o_ref[0] = hb[:, :F2]
    o_ref[1] = hb[:, F2:]


def _mm2_body(a_ref, deg_ref, b_ref, w_ref, o_ref):
    dinv = _dinv_of(deg_ref)
    a = jnp.concatenate([a_ref[0], a_ref[1]], axis=1).astype(jnp.float32)
    z = jnp.maximum(a * dinv + b_ref[...], 0.0)
    h = jnp.dot(z, w_ref[...], preferred_element_type=jnp.float32) * dinv
    hb = h.astype(jnp.bfloat16)
    o_ref[0] = hb[:, :F2]
    o_ref[1] = hb[:, F2:]


def _fin_body(a_ref, deg_ref, b_ref, o_ref):
    dinv = _dinv_of(deg_ref)
    a = jnp.concatenate([a_ref[0], a_ref[1]], axis=1).astype(jnp.float32)
    o_ref[...] = jnp.maximum(a * dinv + b_ref[...], 0.0)


_rows_spec = pl.BlockSpec((ROWBLK, F), lambda i: (i, 0))
_half_spec = pl.BlockSpec((NC, ROWBLK, F2), lambda i: (0, i, 0))
_deg_spec = pl.BlockSpec((NC, ROWBLK, 1), lambda i: (0, i, 0))
_w_spec = pl.BlockSpec((F, F), lambda i: (0, 0))
_b_spec = pl.BlockSpec((1, F), lambda i: (0, 0))
_out_rows = jax.ShapeDtypeStruct((N, F), jnp.float32)
_out_half = jax.ShapeDtypeStruct((NC, N, F2), jnp.bfloat16)

_mma = pl.pallas_call(
    _mma_body, grid=(GRID,),
    in_specs=[_rows_spec, _w_spec],
    out_specs=_rows_spec, out_shape=_out_rows)

_scale = pl.pallas_call(
    _scale_body, grid=(GRID,),
    in_specs=[_rows_spec, _deg_spec],
    out_specs=_half_spec, out_shape=_out_half)

_mm2 = pl.pallas_call(
    _mm2_body, grid=(GRID,),
    in_specs=[_half_spec, _deg_spec, _b_spec, _w_spec],
    out_specs=_half_spec, out_shape=_out_half)

_fin = pl.pallas_call(
    _fin_body, grid=(GRID,),
    in_specs=[_half_spec, _deg_spec, _b_spec],
    out_specs=_rows_spec, out_shape=_out_rows)


def kernel(x, edge_index, W1, b1, W2, b2):
    srcf = jnp.concatenate([edge_index[0], _TAIL_SRC])
    dstf = jnp.concatenate([edge_index[1], _TAIL_DST])
    src_a = srcf.reshape(NS, CH, CHB)
    dst_a = dstf.reshape(NS, CH, CHB)
    dst_d = dstf.reshape(NW, CHD, CHB)
    zeros = jnp.zeros((ZONE, F2), jnp.bfloat16)

    deg_kernel, agg_kernel = _sc_kernels()
    xw1 = _mma(x, W1)
    deg = deg_kernel(dst_d).reshape(NC, NP, 1)[:, :N]
    h1 = _scale(xw1, deg)
    agg1 = agg_kernel(h1, src_a, dst_a, zeros)
    h2 = _mm2(agg1, deg, b1.reshape(1, F), W2)
    agg2 = agg_kernel(h2, src_a, dst_a, zeros)
    return _fin(agg2, deg, b2.reshape(1, F))
